# baseline (speedup 1.0000x reference)
import jax
import jax.numpy as jnp
from jax import lax
from jax.experimental import pallas as pl
from jax.experimental.pallas import tpu as pltpu

_B, _N, _E, _H, _I, _L = 4, 1024, 128, 16, 512, 2
_DH = _E // _H
_BQ = 256
_NQB = _N // _BQ
_SCALE = 1.0 / (_DH ** 0.5)
_NEG = -1e30


def _layer_norm(x, s, b):
    m = jnp.mean(x, axis=0, keepdims=True)
    v = jnp.mean((x - m) * (x - m), axis=0, keepdims=True)
    return (x - m) * lax.rsqrt(v + 1e-5) * s + b


def _gat_kernel(nft_ref, adjt_ref, wq_ref, bq_ref, wk_ref, bk_ref, wv_ref,
                bv_ref, wo_ref, bo_ref, ln1s_ref, ln1b_ref, w1_ref, b1_ref,
                w2_ref, ln2s_ref, ln2b_ref, out_ref, qt, kt, vt, at, mb):
    ht = nft_ref[0]

    aft = adjt_ref[0].astype(jnp.float32)
    mb[...] = aft * (-_NEG) + _NEG
    ones_row = jnp.ones((1, _N), dtype=jnp.float32)
    deg = lax.dot_general(ones_row, aft, (((1,), (0,)), ((), ())))
    indt_all = jnp.where(deg > 0.0, 1.0, 0.0)

    ones_bf = jnp.ones((1, _N), dtype=jnp.bfloat16)
    f32 = jnp.float32

    for l in range(_L):
        qt[...] = jnp.dot(wq_ref[l], ht) + bq_ref[l]
        kt[...] = jnp.dot(wk_ref[l], ht) + bk_ref[l]
        vt[...] = jnp.dot(wv_ref[l], ht) + bv_ref[l]

        def _attn_head(hd, _):
            kh = kt[pl.ds(hd * _DH, _DH), :]
            vh = vt[pl.ds(hd * _DH, _DH), :].astype(jnp.bfloat16)
            kht = lax.transpose(kh, (1, 0))
            for qb in range(_NQB):
                ql = qb * _BQ
                qh = qt[pl.ds(hd * _DH, _DH), ql:ql + _BQ]
                st = lax.dot_general(kht, qh, (((1,), (0,)), ((), ())))
                st = st + mb[:, ql:ql + _BQ]
                et = jnp.exp2(st - jnp.max(st, axis=0, keepdims=True))
                etb = et.astype(jnp.bfloat16)
                denom = lax.dot_general(ones_bf, etb,
                                        (((1,), (0,)), ((), ())),
                                        preferred_element_type=f32)
                ot = lax.dot_general(vh, etb, (((1,), (0,)), ((), ())),
                                     preferred_element_type=f32)
                rscale = indt_all[:, ql:ql + _BQ] / denom
                at[pl.ds(hd * _DH, _DH), ql:ql + _BQ] = ot * rscale
            return 0

        lax.fori_loop(0, _H, _attn_head, 0, unroll=2)

        ao = jnp.dot(wo_ref[l], at[...]) + bo_ref[l]
        h1 = _layer_norm(ht + ao, ln1s_ref[l], ln1b_ref[l])
        ff = jnp.maximum(jnp.dot(w1_ref[l], h1) + b1_ref[l], 0.0)
        f2 = jnp.dot(w2_ref[l], ff)
        ht = _layer_norm(h1 + f2, ln2s_ref[l], ln2b_ref[l])

    out_ref[0] = ht


def _whole(shape):
    return pl.BlockSpec(shape, lambda b: (0,) * len(shape))


def kernel(node_features, batched_adj, Wq, bq, Wk, bk, Wv, bv, Wo, bo,
           ln1_s, ln1_b, W1, b1, W2, ln2_s, ln2_b):
    nft = node_features.transpose(0, 2, 1)
    adj = batched_adj.transpose(0, 2, 1).astype(jnp.int8)
    _qs = _SCALE * 1.4426950408889634
    wqt = Wq.transpose(0, 2, 1) * _qs
    bq = bq * _qs
    wkt = Wk.transpose(0, 2, 1)
    wvt = Wv.transpose(0, 2, 1)
    wot = Wo.transpose(0, 2, 1)
    w1t = W1.transpose(0, 2, 1)
    w2t = W2.transpose(0, 2, 1)
    bqc, bkc, bvc, boc = (x[:, :, None] for x in (bq, bk, bv, bo))
    b1c = b1[:, :, None]
    ln1sc, ln1bc = ln1_s[:, :, None], ln1_b[:, :, None]
    ln2sc, ln2bc = ln2_s[:, :, None], ln2_b[:, :, None]

    outt = pl.pallas_call(
        _gat_kernel,
        grid=(_B,),
        in_specs=[
            pl.BlockSpec((1, _E, _N), lambda b: (b, 0, 0)),
            pl.BlockSpec((1, _N, _N), lambda b: (b, 0, 0)),
            _whole((_L, _E, _E)),
            _whole((_L, _E, 1)),
            _whole((_L, _E, _E)),
            _whole((_L, _E, 1)),
            _whole((_L, _E, _E)),
            _whole((_L, _E, 1)),
            _whole((_L, _E, _E)),
            _whole((_L, _E, 1)),
            _whole((_L, _E, 1)),
            _whole((_L, _E, 1)),
            _whole((_L, _I, _E)),
            _whole((_L, _I, 1)),
            _whole((_L, _E, _I)),
            _whole((_L, _E, 1)),
            _whole((_L, _E, 1)),
        ],
        out_specs=pl.BlockSpec((1, _E, _N), lambda b: (b, 0, 0)),
        out_shape=jax.ShapeDtypeStruct((_B, _E, _N), jnp.float32),
        scratch_shapes=[pltpu.VMEM((_E, _N), jnp.float32)] * 4
        + [pltpu.VMEM((_N, _N), jnp.float32)],
        compiler_params=pltpu.CompilerParams(
            dimension_semantics=("parallel",)),
    )(nft, adj, wqt, bqc, wkt, bkc, wvt, bvc, wot, boc, ln1sc, ln1bc,
      w1t, b1c, w2t, ln2sc, ln2bc)
    return outt.transpose(0, 2, 1)


# speedup vs baseline: 1.0728x; 1.0728x over previous
import jax
import jax.numpy as jnp
from jax import lax
from jax.experimental import pallas as pl
from jax.experimental.pallas import tpu as pltpu

_B, _N, _E, _H, _I, _L = 4, 1024, 128, 16, 512, 2
_DH = _E // _H
_BQ = 256
_NQB = _N // _BQ
_SCALE = 1.0 / (_DH ** 0.5)
_NEG = -1e30


def _layer_norm(x, s, b):
    m = jnp.mean(x, axis=0, keepdims=True)
    v = jnp.mean((x - m) * (x - m), axis=0, keepdims=True)
    return (x - m) * lax.rsqrt(v + 1e-5) * s + b


def _gat_kernel(nft_ref, adjt_ref, wq_ref, bq_ref, wk_ref, bk_ref, wv_ref,
                bv_ref, wo_ref, bo_ref, ln1s_ref, ln1b_ref, w1_ref, b1_ref,
                w2_ref, ln2s_ref, ln2b_ref, out_ref, qt, kt, vt, at, mb):
    ht = nft_ref[0]

    aft = adjt_ref[0].astype(jnp.float32)
    mb[...] = aft * (-_NEG) + _NEG
    ones_row = jnp.ones((1, _N), dtype=jnp.float32)
    deg = lax.dot_general(ones_row, aft, (((1,), (0,)), ((), ())))
    indt_all = jnp.where(deg > 0.0, 1.0, 0.0)

    ones_bf = jnp.ones((1, _N), dtype=jnp.bfloat16)
    f32 = jnp.float32

    for l in range(_L):
        qt[...] = jnp.dot(wq_ref[l], ht) + bq_ref[l]
        kt[...] = jnp.dot(wk_ref[l], ht) + bk_ref[l]
        vt[...] = jnp.dot(wv_ref[l], ht) + bv_ref[l]

        def _attn_head(hd, _):
            kh = kt[pl.ds(hd * _DH, _DH), :]
            vh = vt[pl.ds(hd * _DH, _DH), :].astype(jnp.bfloat16)
            kht = lax.transpose(kh, (1, 0))
            for qb in range(_NQB):
                ql = qb * _BQ
                qh = qt[pl.ds(hd * _DH, _DH), ql:ql + _BQ]
                st = lax.dot_general(kht, qh, (((1,), (0,)), ((), ())))
                st = st + mb[:, ql:ql + _BQ]
                et = jnp.exp2(st - jnp.max(st, axis=0, keepdims=True))
                etb = et.astype(jnp.bfloat16)
                denom = lax.dot_general(ones_bf, etb,
                                        (((1,), (0,)), ((), ())),
                                        preferred_element_type=f32)
                ot = lax.dot_general(vh, etb, (((1,), (0,)), ((), ())),
                                     preferred_element_type=f32)
                rscale = indt_all[:, ql:ql + _BQ] / denom
                at[pl.ds(hd * _DH, _DH), ql:ql + _BQ] = ot * rscale
            return 0

        lax.fori_loop(0, _H, _attn_head, 0, unroll=4)

        ao = jnp.dot(wo_ref[l], at[...]) + bo_ref[l]
        h1 = _layer_norm(ht + ao, ln1s_ref[l], ln1b_ref[l])
        ff = jnp.maximum(jnp.dot(w1_ref[l], h1) + b1_ref[l], 0.0)
        f2 = jnp.dot(w2_ref[l], ff)
        ht = _layer_norm(h1 + f2, ln2s_ref[l], ln2b_ref[l])

    out_ref[0] = ht


def _whole(shape):
    return pl.BlockSpec(shape, lambda b: (0,) * len(shape))


def kernel(node_features, batched_adj, Wq, bq, Wk, bk, Wv, bv, Wo, bo,
           ln1_s, ln1_b, W1, b1, W2, ln2_s, ln2_b):
    nft = node_features.transpose(0, 2, 1)
    adj = batched_adj.transpose(0, 2, 1).astype(jnp.int8)
    _qs = _SCALE * 1.4426950408889634
    wqt = Wq.transpose(0, 2, 1) * _qs
    bq = bq * _qs
    wkt = Wk.transpose(0, 2, 1)
    wvt = Wv.transpose(0, 2, 1)
    wot = Wo.transpose(0, 2, 1)
    w1t = W1.transpose(0, 2, 1)
    w2t = W2.transpose(0, 2, 1)
    bqc, bkc, bvc, boc = (x[:, :, None] for x in (bq, bk, bv, bo))
    b1c = b1[:, :, None]
    ln1sc, ln1bc = ln1_s[:, :, None], ln1_b[:, :, None]
    ln2sc, ln2bc = ln2_s[:, :, None], ln2_b[:, :, None]

    outt = pl.pallas_call(
        _gat_kernel,
        grid=(_B,),
        in_specs=[
            pl.BlockSpec((1, _E, _N), lambda b: (b, 0, 0)),
            pl.BlockSpec((1, _N, _N), lambda b: (b, 0, 0)),
            _whole((_L, _E, _E)),
            _whole((_L, _E, 1)),
            _whole((_L, _E, _E)),
            _whole((_L, _E, 1)),
            _whole((_L, _E, _E)),
            _whole((_L, _E, 1)),
            _whole((_L, _E, _E)),
            _whole((_L, _E, 1)),
            _whole((_L, _E, 1)),
            _whole((_L, _E, 1)),
            _whole((_L, _I, _E)),
            _whole((_L, _I, 1)),
            _whole((_L, _E, _I)),
            _whole((_L, _E, 1)),
            _whole((_L, _E, 1)),
        ],
        out_specs=pl.BlockSpec((1, _E, _N), lambda b: (b, 0, 0)),
        out_shape=jax.ShapeDtypeStruct((_B, _E, _N), jnp.float32),
        scratch_shapes=[pltpu.VMEM((_E, _N), jnp.float32)] * 4
        + [pltpu.VMEM((_N, _N), jnp.float32)],
        compiler_params=pltpu.CompilerParams(
            dimension_semantics=("parallel",)),
    )(nft, adj, wqt, bqc, wkt, bkc, wvt, bvc, wot, boc, ln1sc, ln1bc,
      w1t, b1c, w2t, ln2sc, ln2bc)
    return outt.transpose(0, 2, 1)


# speedup vs baseline: 1.1270x; 1.0506x over previous
import jax
import jax.numpy as jnp
from jax import lax
from jax.experimental import pallas as pl
from jax.experimental.pallas import tpu as pltpu

_B, _N, _E, _H, _I, _L = 4, 1024, 128, 16, 512, 2
_DH = _E // _H
_BQ = 256
_NQB = _N // _BQ
_SCALE = 1.0 / (_DH ** 0.5)
_NEG = -1e30


def _layer_norm(x, s, b):
    m = jnp.mean(x, axis=0, keepdims=True)
    v = jnp.mean((x - m) * (x - m), axis=0, keepdims=True)
    return (x - m) * lax.rsqrt(v + 1e-5) * s + b


def _gat_kernel(nft_ref, adjt_ref, wq_ref, bq_ref, wk_ref, bk_ref, wv_ref,
                bv_ref, wo_ref, bo_ref, ln1s_ref, ln1b_ref, w1_ref, b1_ref,
                w2_ref, ln2s_ref, ln2b_ref, out_ref, qt, kt, vt, at, mb):
    ht = nft_ref[0]

    aft = adjt_ref[0].astype(jnp.float32)
    mb[...] = aft * (-_NEG) + _NEG
    ones_row = jnp.ones((1, _N), dtype=jnp.float32)
    deg = lax.dot_general(ones_row, aft, (((1,), (0,)), ((), ())))
    indt_all = jnp.where(deg > 0.0, 1.0, 0.0)

    ones_bf = jnp.ones((1, _N), dtype=jnp.bfloat16)
    f32 = jnp.float32

    for l in range(_L):
        qt[...] = jnp.dot(wq_ref[l], ht) + bq_ref[l]
        kt[...] = jnp.dot(wk_ref[l], ht) + bk_ref[l]
        vt[...] = jnp.dot(wv_ref[l], ht) + bv_ref[l]

        def _attn_head(hd, _):
            kh = kt[pl.ds(hd * _DH, _DH), :]
            vh = vt[pl.ds(hd * _DH, _DH), :].astype(jnp.bfloat16)
            kht = lax.transpose(kh, (1, 0))
            for qb in range(_NQB):
                ql = qb * _BQ
                qh = qt[pl.ds(hd * _DH, _DH), ql:ql + _BQ]
                st = lax.dot_general(kht, qh, (((1,), (0,)), ((), ())))
                st = st + mb[:, ql:ql + _BQ]
                et = jnp.exp2(st - jnp.max(st, axis=0, keepdims=True))
                etb = et.astype(jnp.bfloat16)
                denom = lax.dot_general(ones_bf, etb,
                                        (((1,), (0,)), ((), ())),
                                        preferred_element_type=f32)
                ot = lax.dot_general(vh, etb, (((1,), (0,)), ((), ())),
                                     preferred_element_type=f32)
                rscale = indt_all[:, ql:ql + _BQ] / denom
                at[pl.ds(hd * _DH, _DH), ql:ql + _BQ] = ot * rscale
            return 0

        lax.fori_loop(0, _H, _attn_head, 0, unroll=8)

        ao = jnp.dot(wo_ref[l], at[...]) + bo_ref[l]
        h1 = _layer_norm(ht + ao, ln1s_ref[l], ln1b_ref[l])
        ff = jnp.maximum(jnp.dot(w1_ref[l], h1) + b1_ref[l], 0.0)
        f2 = jnp.dot(w2_ref[l], ff)
        ht = _layer_norm(h1 + f2, ln2s_ref[l], ln2b_ref[l])

    out_ref[0] = ht


def _whole(shape):
    return pl.BlockSpec(shape, lambda b: (0,) * len(shape))


def kernel(node_features, batched_adj, Wq, bq, Wk, bk, Wv, bv, Wo, bo,
           ln1_s, ln1_b, W1, b1, W2, ln2_s, ln2_b):
    nft = node_features.transpose(0, 2, 1)
    adj = batched_adj.transpose(0, 2, 1).astype(jnp.int8)
    _qs = _SCALE * 1.4426950408889634
    wqt = Wq.transpose(0, 2, 1) * _qs
    bq = bq * _qs
    wkt = Wk.transpose(0, 2, 1)
    wvt = Wv.transpose(0, 2, 1)
    wot = Wo.transpose(0, 2, 1)
    w1t = W1.transpose(0, 2, 1)
    w2t = W2.transpose(0, 2, 1)
    bqc, bkc, bvc, boc = (x[:, :, None] for x in (bq, bk, bv, bo))
    b1c = b1[:, :, None]
    ln1sc, ln1bc = ln1_s[:, :, None], ln1_b[:, :, None]
    ln2sc, ln2bc = ln2_s[:, :, None], ln2_b[:, :, None]

    outt = pl.pallas_call(
        _gat_kernel,
        grid=(_B,),
        in_specs=[
            pl.BlockSpec((1, _E, _N), lambda b: (b, 0, 0)),
            pl.BlockSpec((1, _N, _N), lambda b: (b, 0, 0)),
            _whole((_L, _E, _E)),
            _whole((_L, _E, 1)),
            _whole((_L, _E, _E)),
            _whole((_L, _E, 1)),
            _whole((_L, _E, _E)),
            _whole((_L, _E, 1)),
            _whole((_L, _E, _E)),
            _whole((_L, _E, 1)),
            _whole((_L, _E, 1)),
            _whole((_L, _E, 1)),
            _whole((_L, _I, _E)),
            _whole((_L, _I, 1)),
            _whole((_L, _E, _I)),
            _whole((_L, _E, 1)),
            _whole((_L, _E, 1)),
        ],
        out_specs=pl.BlockSpec((1, _E, _N), lambda b: (b, 0, 0)),
        out_shape=jax.ShapeDtypeStruct((_B, _E, _N), jnp.float32),
        scratch_shapes=[pltpu.VMEM((_E, _N), jnp.float32)] * 4
        + [pltpu.VMEM((_N, _N), jnp.float32)],
        compiler_params=pltpu.CompilerParams(
            dimension_semantics=("parallel",)),
    )(nft, adj, wqt, bqc, wkt, bkc, wvt, bvc, wot, boc, ln1sc, ln1bc,
      w1t, b1c, w2t, ln2sc, ln2bc)
    return outt.transpose(0, 2, 1)
